# baseline (speedup 1.0000x reference)
import functools

import jax
import jax.numpy as jnp
from jax.experimental import pallas as pl
from jax.experimental.pallas import tpu as pltpu

_EPS = 1e-5


def _fused_encoder(x_ref, m_ref, wpre_ref, spre_ref, bpre_ref,
                   w1a_ref, w1b_ref, s1_ref, b1_ref,
                   w2_ref, s2_ref, b2_ref,
                   wout_ref, bout_ref, out_ref, *, n_pts, tile):
    rows = n_pts * tile
    c = x_ref.shape[-1]
    x = x_ref[...].reshape(rows, c)
    m = m_ref[...].reshape(rows, 1)
    h = jnp.maximum(
        jnp.dot(x, wpre_ref[...], preferred_element_type=jnp.float32)
        * spre_ref[...] + bpre_ref[...], 0.0) * m
    pooled = jnp.max(h.reshape(n_pts, tile, -1), axis=0)
    z = jnp.dot(h, w1a_ref[...], preferred_element_type=jnp.float32)
    z = (z.reshape(n_pts, tile, -1)
         + jnp.dot(pooled, w1b_ref[...], preferred_element_type=jnp.float32)[None])
    h2 = jnp.maximum(z.reshape(rows, -1) * s1_ref[...] + b1_ref[...], 0.0)
    h2 = jnp.maximum(
        jnp.dot(h2, w2_ref[...], preferred_element_type=jnp.float32)
        * s2_ref[...] + b2_ref[...], 0.0) * m
    poly = jnp.max(h2.reshape(n_pts, tile, -1), axis=0)
    valid = jnp.max(m.reshape(n_pts, tile, 1), axis=0)
    out_ref[...] = (
        jnp.dot(poly, wout_ref[...], preferred_element_type=jnp.float32)
        + bout_ref[...]) * valid


def kernel(polylines, polylines_mask, W_pre, g_pre, b_pre,
           W1, g1, b1, W2, g2, b2, W_out, b_out):
    B, P, N, C = polylines.shape
    H = W_pre.shape[1]
    O = W_out.shape[1]
    M = B * P
    tile = 128
    grid = M // tile

    xt = polylines.reshape(M, N, C).transpose(1, 0, 2)
    mt = polylines_mask.reshape(M, N).T.astype(jnp.float32)[..., None]

    inv = 1.0 / jnp.sqrt(1.0 + _EPS)
    spre = (g_pre * inv).reshape(1, H)
    s1 = (g1 * inv).reshape(1, H)
    s2 = (g2 * inv).reshape(1, H)

    out = pl.pallas_call(
        functools.partial(_fused_encoder, n_pts=N, tile=tile),
        grid=(grid,),
        in_specs=[
            pl.BlockSpec((N, tile, C), lambda i: (0, i, 0)),
            pl.BlockSpec((N, tile, 1), lambda i: (0, i, 0)),
            pl.BlockSpec((C, H), lambda i: (0, 0)),
            pl.BlockSpec((1, H), lambda i: (0, 0)),
            pl.BlockSpec((1, H), lambda i: (0, 0)),
            pl.BlockSpec((H, H), lambda i: (0, 0)),
            pl.BlockSpec((H, H), lambda i: (0, 0)),
            pl.BlockSpec((1, H), lambda i: (0, 0)),
            pl.BlockSpec((1, H), lambda i: (0, 0)),
            pl.BlockSpec((H, H), lambda i: (0, 0)),
            pl.BlockSpec((1, H), lambda i: (0, 0)),
            pl.BlockSpec((1, H), lambda i: (0, 0)),
            pl.BlockSpec((H, O), lambda i: (0, 0)),
            pl.BlockSpec((1, O), lambda i: (0, 0)),
        ],
        out_specs=pl.BlockSpec((tile, O), lambda i: (i, 0)),
        out_shape=jax.ShapeDtypeStruct((M, O), jnp.float32),
        compiler_params=pltpu.CompilerParams(
            dimension_semantics=("parallel",)),
    )(xt, mt, W_pre, spre, b_pre.reshape(1, H),
      W1[:H], W1[H:], s1, b1.reshape(1, H),
      W2, s2, b2.reshape(1, H),
      W_out, b_out.reshape(1, O))
    return out.reshape(B, P, O)


# speedup vs baseline: 1.0167x; 1.0167x over previous
import functools

import jax
import jax.numpy as jnp
from jax.experimental import pallas as pl
from jax.experimental.pallas import tpu as pltpu

_EPS = 1e-5


def _fused_encoder(x_ref, m_ref, wpre_ref, bpre_ref,
                   w1a_ref, w1b_ref, b1_ref,
                   w2_ref, b2_ref,
                   wout_ref, bout_ref, out_ref, *, n_pts, tile):
    rows = n_pts * tile
    c = x_ref.shape[-1]
    x = x_ref[...].reshape(rows, c)
    m = m_ref[...].reshape(rows, 1)
    h = jnp.maximum(
        jnp.dot(x, wpre_ref[...], preferred_element_type=jnp.float32)
        + bpre_ref[...], 0.0) * m
    pooled = jnp.max(h.reshape(n_pts, tile, -1), axis=0)
    z = jnp.dot(h, w1a_ref[...], preferred_element_type=jnp.float32)
    z = (z.reshape(n_pts, tile, -1)
         + jnp.dot(pooled, w1b_ref[...], preferred_element_type=jnp.float32)[None])
    h2 = jnp.maximum(z.reshape(rows, -1) + b1_ref[...], 0.0)
    h2 = jnp.maximum(
        jnp.dot(h2, w2_ref[...], preferred_element_type=jnp.float32)
        + b2_ref[...], 0.0) * m
    poly = jnp.max(h2.reshape(n_pts, tile, -1), axis=0)
    valid = jnp.max(m.reshape(n_pts, tile, 1), axis=0)
    out_ref[...] = (
        jnp.dot(poly, wout_ref[...], preferred_element_type=jnp.float32)
        + bout_ref[...]) * valid


def kernel(polylines, polylines_mask, W_pre, g_pre, b_pre,
           W1, g1, b1, W2, g2, b2, W_out, b_out):
    B, P, N, C = polylines.shape
    H = W_pre.shape[1]
    O = W_out.shape[1]
    M = B * P
    tile = 128
    grid = M // tile

    xt = polylines.reshape(M, N, C).transpose(1, 0, 2)
    mt = polylines_mask.reshape(M, N).T.astype(jnp.float32)[..., None]

    inv = 1.0 / jnp.sqrt(1.0 + _EPS)
    wpre = W_pre * (g_pre * inv)
    w1 = W1 * (g1 * inv)
    w2 = W2 * (g2 * inv)

    out = pl.pallas_call(
        functools.partial(_fused_encoder, n_pts=N, tile=tile),
        grid=(grid,),
        in_specs=[
            pl.BlockSpec((N, tile, C), lambda i: (0, i, 0)),
            pl.BlockSpec((N, tile, 1), lambda i: (0, i, 0)),
            pl.BlockSpec((C, H), lambda i: (0, 0)),
            pl.BlockSpec((1, H), lambda i: (0, 0)),
            pl.BlockSpec((H, H), lambda i: (0, 0)),
            pl.BlockSpec((H, H), lambda i: (0, 0)),
            pl.BlockSpec((1, H), lambda i: (0, 0)),
            pl.BlockSpec((H, H), lambda i: (0, 0)),
            pl.BlockSpec((1, H), lambda i: (0, 0)),
            pl.BlockSpec((H, O), lambda i: (0, 0)),
            pl.BlockSpec((1, O), lambda i: (0, 0)),
        ],
        out_specs=pl.BlockSpec((tile, O), lambda i: (i, 0)),
        out_shape=jax.ShapeDtypeStruct((M, O), jnp.float32),
        compiler_params=pltpu.CompilerParams(
            dimension_semantics=("parallel",)),
    )(xt, mt, wpre, b_pre.reshape(1, H),
      w1[:H], w1[H:], b1.reshape(1, H),
      w2, b2.reshape(1, H),
      W_out, b_out.reshape(1, O))
    return out.reshape(B, P, O)
